# baseline (speedup 1.0000x reference)
import jax
import jax.numpy as jnp
from jax import lax
from jax.experimental import pallas as pl
from jax.experimental.pallas import tpu as pltpu
from jax.experimental.pallas import tpu_sc as plsc

N = 10000
E = 320000
F = 128
FH = F // 2
ALPHA = 0.2

NC = 2
NS = 16
NT = NC * NS
EC = E // NT
ECS = E // NS
B = 80
NBUF = 3
NBUFS = 2
NP = 10240
ROWS_PER_TILE = NP // NS
DR = NP // 16
DRT = DR // NS

MM_BLK = 1000
MM_GRID = N // MM_BLK


def _mm_body(x_ref, w_ref, a_ref, wh_ref, s1_ref, s2_ref):
    xb = x_ref[...]
    wh = jnp.dot(xb, w_ref[...], preferred_element_type=jnp.float32)
    wh_ref[0] = wh[:, :FH].astype(jnp.bfloat16)
    wh_ref[1] = wh[:, FH:].astype(jnp.bfloat16)
    s1_ref[...] = jnp.dot(wh, a_ref[:F], preferred_element_type=jnp.float32)
    s2_ref[...] = jnp.dot(wh, a_ref[F:], preferred_element_type=jnp.float32)


def _stage1(x, W, a):
    return pl.pallas_call(
        _mm_body,
        grid=(MM_GRID,),
        in_specs=[
            pl.BlockSpec((MM_BLK, F), lambda i: (i, 0)),
            pl.BlockSpec((F, F), lambda i: (0, 0)),
            pl.BlockSpec((2 * F, 1), lambda i: (0, 0)),
        ],
        out_specs=[
            pl.BlockSpec((NC, MM_BLK, FH), lambda i: (0, i, 0)),
            pl.BlockSpec((MM_BLK, 1), lambda i: (i, 0)),
            pl.BlockSpec((MM_BLK, 1), lambda i: (i, 0)),
        ],
        out_shape=[
            jax.ShapeDtypeStruct((NC, N, FH), jnp.bfloat16),
            jax.ShapeDtypeStruct((N, 1), jnp.float32),
            jax.ShapeDtypeStruct((N, 1), jnp.float32),
        ],
    )(x, W, a)


def _sc_score_body(s1_hbm, s2_hbm, src_hbm, dst_hbm,
                   ex_out, den_out,
                   s1_v, s2_v, src_v, dst_v, ex_v, den_v):
    c = lax.axis_index("c")
    s = lax.axis_index("s")
    g = c * NS + s
    base = g * EC

    pltpu.sync_copy(s1_hbm, s1_v)
    pltpu.sync_copy(s2_hbm, s2_v)
    pltpu.sync_copy(src_hbm.at[pl.ds(base, EC)], src_v)
    pltpu.sync_copy(dst_hbm.at[pl.ds(base, EC)], dst_v)

    zv = jnp.zeros((16,), jnp.float32)

    def _zden(i, _):
        den_v[pl.ds(i * 16, 16)] = zv
        return 0
    lax.fori_loop(0, NP // 16, _zden, 0)

    def _score(i, _):
        d16 = dst_v[pl.ds(i * 16, 16)]
        t16 = src_v[pl.ds(i * 16, 16)]
        e = plsc.load_gather(s1_v, [d16]) + plsc.load_gather(s2_v, [t16])
        e = jnp.where(e >= 0.0, e, ALPHA * e)
        ex = jnp.exp(e)
        ex_v[pl.ds(i * 16, 16)] = ex
        plsc.addupdate_scatter(den_v, [d16], ex)
        return 0
    lax.fori_loop(0, EC // 16, _score, 0)

    pltpu.sync_copy(ex_v, ex_out.at[pl.ds(base, EC)])
    pltpu.sync_copy(den_v, den_out.at[pl.ds(g * NP, NP)])


def _stage2a(s1, s2, src, dst):
    mesh = plsc.VectorSubcoreMesh(core_axis_name="c", subcore_axis_name="s",
                                  num_cores=NC, num_subcores=NS)
    return pl.kernel(
        _sc_score_body,
        out_type=[
            jax.ShapeDtypeStruct((E,), jnp.float32),
            jax.ShapeDtypeStruct((NT * NP,), jnp.float32),
        ],
        mesh=mesh,
        compiler_params=pltpu.CompilerParams(needs_layout_passes=False),
        scratch_types=[
            pltpu.VMEM((N,), jnp.float32),
            pltpu.VMEM((N,), jnp.float32),
            pltpu.VMEM((EC,), jnp.int32),
            pltpu.VMEM((EC,), jnp.int32),
            pltpu.VMEM((EC,), jnp.float32),
            pltpu.VMEM((NP,), jnp.float32),
        ],
    )(s1, s2, src, dst)


CH = 2000
NCH = ECS // CH
NBC = CH // B
FB = ROWS_PER_TILE // B


def _sc_rows_body(wh_hbm, src_hbm, dstw_hbm, ex_hbm, den_hbm,
                  out_hbm,
                  src_c, dstw_c, ex_c, rows_bf, rows_v, den_w, rec_w,
                  sems, ssems, dsems,
                  acc_s):
    c = lax.axis_index("c")
    s = lax.axis_index("s")
    base = s * ECS
    row0 = s * ROWS_PER_TILE

    zv = jnp.zeros((16,), jnp.float32)

    def _zrow(r, _):
        for f in range(FH // 16):
            rows_v[0, r, pl.ds(f * 16, 16)] = zv
        return 0
    lax.fori_loop(0, B, _zrow, 0)
    for t in range(FB):
        pltpu.sync_copy(rows_v.at[0], acc_s.at[pl.ds(row0 + t * B, B)])

    plsc.subcore_barrier()

    def _issue(b, k):
        idx = src_c.at[pl.ds(b * B, B)]
        pltpu.async_copy(wh_hbm.at[c].at[idx], rows_bf.at[k], sems.at[k])

    def _scale_rows(kb, ks, b):
        @plsc.parallel_loop(0, B, unroll=8)
        def _edge(kk):
            eidx = b * B + kk
            exb = plsc.load_gather(ex_c, [jnp.full((16,), eidx, jnp.int32)])
            for f in range(FH // 32):
                x = plsc.bitcast(rows_bf[kb, kk, pl.ds(f * 16, 16)],
                                 jnp.bfloat16)
                lo, hi = plsc.unpack(x, format=plsc.PackFormat.INTERLEAVED)
                rows_v[ks, kk, pl.ds(f * 32, 16)] = lo * exb
                rows_v[ks, kk, pl.ds(f * 32 + 16, 16)] = hi * exb

    def _scatter(k, b):
        return pltpu.make_async_copy(rows_v.at[k], acc_s.at[dstw_c.at[b]],
                                     ssems.at[k])

    def _chunk(ch, _):
        pltpu.sync_copy(src_hbm.at[pl.ds(base + ch * CH, CH)], src_c)
        pltpu.sync_copy(ex_hbm.at[pl.ds(base + ch * CH, CH)], ex_c)
        pltpu.sync_copy(dstw_hbm.at[s, ch], dstw_c)

        for k in range(NBUF):
            _issue(k, k)

        LCM = NBUF * NBUFS if NBUF % NBUFS else NBUF

        def _outer(bb, _):
            for k in range(LCM):
                b = bb * LCM + k
                kb = k % NBUF
                ks = k % NBUFS

                @pl.when(b < NBC)
                def _():
                    pltpu.make_async_copy(
                        wh_hbm.at[c].at[src_c.at[pl.ds(b * B, B)]],
                        rows_bf.at[kb], sems.at[kb]).wait()

                    @pl.when(b >= NBUFS)
                    def _():
                        _scatter(ks, b - NBUFS).wait()
                    _scale_rows(kb, ks, b)
                    _scatter(ks, b).start(add=True)

                    @pl.when(b + NBUF < NBC)
                    def _():
                        _issue(b + NBUF, kb)
            return 0
        lax.fori_loop(0, (NBC + LCM - 1) // LCM, _outer, 0)
        for d in range(NBUFS):
            _scatter((NBC - NBUFS + d) % NBUFS, NBC - NBUFS + d).wait()
        return 0
    lax.fori_loop(0, NCH, _chunk, 0)

    plsc.subcore_barrier()

    def _zrec(i, _):
        rec_w[pl.ds(i * 16, 16)] = zv
        return 0
    lax.fori_loop(0, ROWS_PER_TILE // 16, _zrec, 0)

    def _dget(t2, k):
        pltpu.async_copy(
            den_hbm.at[pl.ds(t2 * NP + s * ROWS_PER_TILE, ROWS_PER_TILE)],
            den_w.at[k], dsems.at[k])

    for k in range(4):
        _dget(k, k)

    def _dacc(t2, _):
        for k in range(4):
            tt = t2 * 4 + k
            pltpu.make_async_copy(
                den_hbm.at[pl.ds(tt * NP + s * ROWS_PER_TILE,
                                 ROWS_PER_TILE)],
                den_w.at[k], dsems.at[k]).wait()

            def _radd(i, _):
                sl = pl.ds(i * 16, 16)
                rec_w[sl] = rec_w[sl] + den_w[k, sl]
                return 0
            lax.fori_loop(0, ROWS_PER_TILE // 16, _radd, 0)

            @pl.when(tt + 4 < NT)
            def _():
                _dget(tt + 4, k)
        return 0
    lax.fori_loop(0, NT // 4, _dacc, 0)

    def _rec(i, _):
        sl = pl.ds(i * 16, 16)
        rec_w[sl] = 1.0 / (rec_w[sl] + 1e-16)
        return 0
    lax.fori_loop(0, ROWS_PER_TILE // 16, _rec, 0)

    for t in range(FB):
        pltpu.sync_copy(acc_s.at[pl.ds(row0 + t * B, B)], rows_v.at[0])

        @plsc.parallel_loop(0, B, unroll=8)
        def _row(r):
            rr = t * B + r
            rec = plsc.load_gather(rec_w, [jnp.full((16,), rr, jnp.int32)])
            for f in range(FH // 16):
                sl = pl.ds(f * 16, 16)
                o = rows_v[0, r, sl] * rec
                e = jnp.exp(jnp.minimum(o, 0.0)) - 1.0
                rows_v[1, r, sl] = jnp.where(o > 0.0, o, e)
        pltpu.sync_copy(rows_v.at[1], out_hbm.at[c, pl.ds(row0 + t * B, B)])


def _stage2b(whpk, src, dstw, ex, den):
    mesh = plsc.VectorSubcoreMesh(core_axis_name="c", subcore_axis_name="s",
                                  num_cores=NC, num_subcores=NS)
    return pl.kernel(
        _sc_rows_body,
        out_type=jax.ShapeDtypeStruct((NC, NP, FH), jnp.float32),
        mesh=mesh,
        compiler_params=pltpu.CompilerParams(needs_layout_passes=False,
                                             use_tc_tiling_on_sc=False),
        scratch_types=[
            pltpu.VMEM((CH,), jnp.int32),
            pltpu.VMEM((NBC, B), jnp.int32),
            pltpu.VMEM((CH,), jnp.float32),
            pltpu.VMEM((NBUF, B, FH // 2), jnp.int32),
            pltpu.VMEM((NBUFS, B, FH), jnp.float32),
            pltpu.VMEM((4, ROWS_PER_TILE), jnp.float32),
            pltpu.VMEM((ROWS_PER_TILE,), jnp.float32),
            pltpu.SemaphoreType.DMA((NBUF,)),
            pltpu.SemaphoreType.DMA((NBUFS,)),
            pltpu.SemaphoreType.DMA((4,)),
            pltpu.VMEM_SHARED((NP, FH), jnp.float32),
        ],
    )(whpk, src, dstw, ex, den)


def kernel(x, edge_index, W, a):
    src = edge_index[0]
    dst = edge_index[1]
    wh_bf, s1, s2 = _stage1(x, W, a)
    wh_bf = (wh_bf.reshape(NC, N, FH // 32, 2, 16)
             .transpose(0, 1, 2, 4, 3).reshape(NC, N, FH))
    whpk = lax.bitcast_convert_type(wh_bf.reshape(NC, N, FH // 2, 2),
                                    jnp.int32)
    ex, den = _stage2a(s1.reshape(N), s2.reshape(N), src, dst)
    dstw = dst.reshape(NS, NCH, NBC, B)
    outp = _stage2b(whpk, src, dstw, ex, den)
    return jnp.concatenate([outp[0, :N], outp[1, :N]], axis=1)


# speedup vs baseline: 1.1608x; 1.1608x over previous
import jax
import jax.numpy as jnp
from jax import lax
from jax.experimental import pallas as pl
from jax.experimental.pallas import tpu as pltpu
from jax.experimental.pallas import tpu_sc as plsc

N = 10000
E = 320000
F = 128
ALPHA = 0.2

NC = 2
NS = 16
NT = NC * NS
EC = E // NT
B = 80
NBUF = 3
NBUFS = 2
NP = 10240
ROWS_PER_TILE = NP // NS

MM_BLK = 1000
MM_GRID = N // MM_BLK
ROW_BLK = 1024
GRID = NP // ROW_BLK


def _mm_body(x_ref, w_ref, a_ref, wh_ref, s1_ref, s2_ref):
    xb = x_ref[...]
    wh = jnp.dot(xb, w_ref[...], preferred_element_type=jnp.float32)
    wh_ref[...] = wh.astype(jnp.bfloat16)
    s1_ref[...] = jnp.dot(wh, a_ref[:F], preferred_element_type=jnp.float32)
    s2_ref[...] = jnp.dot(wh, a_ref[F:], preferred_element_type=jnp.float32)


def _stage1(x, W, a):
    return pl.pallas_call(
        _mm_body,
        grid=(MM_GRID,),
        in_specs=[
            pl.BlockSpec((MM_BLK, F), lambda i: (i, 0)),
            pl.BlockSpec((F, F), lambda i: (0, 0)),
            pl.BlockSpec((2 * F, 1), lambda i: (0, 0)),
        ],
        out_specs=[
            pl.BlockSpec((MM_BLK, F), lambda i: (i, 0)),
            pl.BlockSpec((MM_BLK, 1), lambda i: (i, 0)),
            pl.BlockSpec((MM_BLK, 1), lambda i: (i, 0)),
        ],
        out_shape=[
            jax.ShapeDtypeStruct((N, F), jnp.bfloat16),
            jax.ShapeDtypeStruct((N, 1), jnp.float32),
            jax.ShapeDtypeStruct((N, 1), jnp.float32),
        ],
    )(x, W, a)


def _sc_score_body(s1_hbm, s2_hbm, src_hbm, dst_hbm,
                   ex_out, den_out,
                   s1_v, s2_v, src_v, dst_v, ex_v, den_v):
    c = lax.axis_index("c")
    s = lax.axis_index("s")
    g = c * NS + s
    base = g * EC

    pltpu.sync_copy(s1_hbm, s1_v)
    pltpu.sync_copy(s2_hbm, s2_v)
    pltpu.sync_copy(src_hbm.at[pl.ds(base, EC)], src_v)
    pltpu.sync_copy(dst_hbm.at[pl.ds(base, EC)], dst_v)

    zv = jnp.zeros((16,), jnp.float32)

    def _zden(i, _):
        den_v[pl.ds(i * 16, 16)] = zv
        return 0
    lax.fori_loop(0, NP // 16, _zden, 0)

    def _score(i, _):
        d16 = dst_v[pl.ds(i * 16, 16)]
        t16 = src_v[pl.ds(i * 16, 16)]
        e = plsc.load_gather(s1_v, [d16]) + plsc.load_gather(s2_v, [t16])
        e = jnp.where(e >= 0.0, e, ALPHA * e)
        ex = jnp.exp(e)
        ex_v[pl.ds(i * 16, 16)] = ex
        plsc.addupdate_scatter(den_v, [d16], ex)
        return 0
    lax.fori_loop(0, EC // 16, _score, 0)

    pltpu.sync_copy(ex_v, ex_out.at[pl.ds(base, EC)])
    pltpu.sync_copy(den_v, den_out.at[pl.ds(g * NP, NP)])


def _stage2a(s1, s2, src, dst):
    mesh = plsc.VectorSubcoreMesh(core_axis_name="c", subcore_axis_name="s",
                                  num_cores=NC, num_subcores=NS)
    return pl.kernel(
        _sc_score_body,
        out_type=[
            jax.ShapeDtypeStruct((E,), jnp.float32),
            jax.ShapeDtypeStruct((NT * NP,), jnp.float32),
        ],
        mesh=mesh,
        compiler_params=pltpu.CompilerParams(needs_layout_passes=False),
        scratch_types=[
            pltpu.VMEM((N,), jnp.float32),
            pltpu.VMEM((N,), jnp.float32),
            pltpu.VMEM((EC,), jnp.int32),
            pltpu.VMEM((EC,), jnp.int32),
            pltpu.VMEM((EC,), jnp.float32),
            pltpu.VMEM((NP,), jnp.float32),
        ],
    )(s1, s2, src, dst)


CH = 2000
NCH = EC // CH
NBC = CH // B


def _sc_rows_body(wh_hbm, src_hbm, dstw_hbm, ex_hbm,
                  acc_out,
                  src_c, dstw_c, ex_c, rows_bf, rows_v, sems, ssems,
                  acc_s):
    c = lax.axis_index("c")
    s = lax.axis_index("s")
    g = c * NS + s
    base = g * EC

    zv = jnp.zeros((16,), jnp.float32)

    def _zrow(r, _):
        for f in range(F // 16):
            rows_v[0, r, pl.ds(f * 16, 16)] = zv
        return 0
    lax.fori_loop(0, B, _zrow, 0)
    row0 = s * ROWS_PER_TILE
    for t in range(ROWS_PER_TILE // B):
        pltpu.sync_copy(rows_v.at[0], acc_s.at[pl.ds(row0 + t * B, B)])

    plsc.subcore_barrier()

    def _issue(b, k):
        idx = src_c.at[pl.ds(b * B, B)]
        pltpu.async_copy(wh_hbm.at[idx], rows_bf.at[k], sems.at[k])

    def _scale_rows(kb, ks, b):
        @plsc.parallel_loop(0, B, unroll=8)
        def _edge(kk):
            eidx = b * B + kk
            exb = plsc.load_gather(ex_c, [jnp.full((16,), eidx, jnp.int32)])
            for f in range(F // 32):
                x = plsc.bitcast(rows_bf[kb, kk, pl.ds(f * 16, 16)],
                                 jnp.bfloat16)
                lo, hi = plsc.unpack(x, format=plsc.PackFormat.INTERLEAVED)
                rows_v[ks, kk, pl.ds(f * 32, 16)] = lo * exb
                rows_v[ks, kk, pl.ds(f * 32 + 16, 16)] = hi * exb

    def _scatter(k, b):
        return pltpu.make_async_copy(rows_v.at[k], acc_s.at[dstw_c.at[b]],
                                     ssems.at[k])

    def _chunk(ch, _):
        pltpu.sync_copy(src_hbm.at[pl.ds(base + ch * CH, CH)], src_c)
        pltpu.sync_copy(ex_hbm.at[pl.ds(base + ch * CH, CH)], ex_c)
        pltpu.sync_copy(dstw_hbm.at[g, ch], dstw_c)

        for k in range(NBUF):
            _issue(k, k)

        LCM = NBUF * NBUFS if NBUF % NBUFS else NBUF

        def _outer(bb, _):
            for k in range(LCM):
                b = bb * LCM + k
                kb = k % NBUF
                ks = k % NBUFS

                @pl.when(b < NBC)
                def _():
                    pltpu.make_async_copy(
                        wh_hbm.at[src_c.at[pl.ds(b * B, B)]],
                        rows_bf.at[kb], sems.at[kb]).wait()

                    @pl.when(b >= NBUFS)
                    def _():
                        _scatter(ks, b - NBUFS).wait()
                    _scale_rows(kb, ks, b)
                    _scatter(ks, b).start(add=True)

                    @pl.when(b + NBUF < NBC)
                    def _():
                        _issue(b + NBUF, kb)
            return 0
        lax.fori_loop(0, (NBC + LCM - 1) // LCM, _outer, 0)
        for d in range(NBUFS):
            _scatter((NBC - NBUFS + d) % NBUFS, NBC - NBUFS + d).wait()
        return 0
    lax.fori_loop(0, NCH, _chunk, 0)

    plsc.subcore_barrier()
    pltpu.sync_copy(acc_s.at[pl.ds(row0, ROWS_PER_TILE)],
                    acc_out.at[c, pl.ds(row0, ROWS_PER_TILE)])


def _stage2b(wh, src, dstw, ex):
    mesh = plsc.VectorSubcoreMesh(core_axis_name="c", subcore_axis_name="s",
                                  num_cores=NC, num_subcores=NS)
    return pl.kernel(
        _sc_rows_body,
        out_type=jax.ShapeDtypeStruct((NC, NP, F), jnp.float32),
        mesh=mesh,
        compiler_params=pltpu.CompilerParams(needs_layout_passes=False,
                                             use_tc_tiling_on_sc=False),
        scratch_types=[
            pltpu.VMEM((CH,), jnp.int32),
            pltpu.VMEM((NBC, B), jnp.int32),
            pltpu.VMEM((CH,), jnp.float32),
            pltpu.VMEM((NBUF, B, F // 2), jnp.int32),
            pltpu.VMEM((NBUFS, B, F), jnp.float32),
            pltpu.SemaphoreType.DMA((NBUF,)),
            pltpu.SemaphoreType.DMA((NBUFS,)),
            pltpu.VMEM_SHARED((NP, F), jnp.float32),
        ],
    )(wh, src, dstw, ex)


def _fin_body(acc_ref, den_ref, out_ref):
    i = pl.program_id(0)
    acc = acc_ref[0] + acc_ref[1]
    den = jnp.sum(den_ref[:, pl.ds(i * ROW_BLK, ROW_BLK)], axis=0)
    o = acc / (den[:, None] + 1e-16)
    out_ref[...] = jnp.where(o > 0.0, o, jnp.exp(jnp.minimum(o, 0.0)) - 1.0)


def _stage3(acc, den):
    return pl.pallas_call(
        _fin_body,
        grid=(GRID,),
        in_specs=[
            pl.BlockSpec((NC, ROW_BLK, F), lambda i: (0, i, 0)),
            pl.BlockSpec((NT, NP), lambda i: (0, 0)),
        ],
        out_specs=pl.BlockSpec((ROW_BLK, F), lambda i: (i, 0)),
        out_shape=jax.ShapeDtypeStruct((NP, F), jnp.float32),
    )(acc, den)


def kernel(x, edge_index, W, a):
    src = edge_index[0]
    dst = edge_index[1]
    wh_bf, s1, s2 = _stage1(x, W, a)
    wh_bf = (wh_bf.reshape(N, F // 32, 2, 16)
             .transpose(0, 1, 3, 2).reshape(N, F))
    wh_bf = lax.bitcast_convert_type(wh_bf.reshape(N, F // 2, 2),
                                     jnp.int32)
    ex, den_parts = _stage2a(s1.reshape(N), s2.reshape(N), src, dst)
    dstw = dst.reshape(NT, NCH, NBC, B)
    acc = _stage2b(wh_bf, src, dstw, ex)
    return _stage3(acc, den_parts.reshape(NT, NP))[:N]
